# Initial kernel scaffold; baseline (speedup 1.0000x reference)
import jax
import jax.numpy as jnp
from jax.experimental import pallas as pl


def kernel(x, edge, pool_edge, finer_edge, W, b):
    raise NotImplementedError("write your pallas kernel here")


# speedup vs baseline: 2.6839x; 2.6839x over previous
import functools

import jax
import jax.numpy as jnp
from jax import lax
from jax.experimental import pallas as pl
from jax.experimental.pallas import tpu as pltpu
from jax.experimental.pallas import tpu_sc as plsc

_CH = 128


def _sc_gather(x2d, src):
    B = src.shape[0]
    fv = x2d.shape[1]
    info = plsc.get_sparse_core_info()
    nw = info.num_cores * info.num_subcores
    per_w = B // nw
    n_ch = per_w // _CH
    mesh = plsc.VectorSubcoreMesh(core_axis_name="c", subcore_axis_name="s")

    @functools.partial(
        pl.kernel,
        mesh=mesh,
        out_type=jax.ShapeDtypeStruct((B, fv), jnp.float32),
        scratch_types=[
            pltpu.VMEM((_CH,), jnp.int32),
            pltpu.VMEM((_CH, fv), jnp.float32),
            pltpu.SemaphoreType.DMA,
        ],
    )
    def gather_kernel(x_hbm, src_hbm, out_hbm, idx_v, rows_v, sem):
        wid = lax.axis_index("s") * info.num_cores + lax.axis_index("c")
        base = wid * per_w

        def body(i, carry):
            off = base + i * _CH
            pltpu.sync_copy(src_hbm.at[pl.ds(off, _CH)], idx_v)
            pltpu.async_copy(x_hbm.at[idx_v], rows_v, sem).wait()
            pltpu.sync_copy(rows_v, out_hbm.at[pl.ds(off, _CH)])
            return carry

        lax.fori_loop(0, n_ch, body, 0)

    return gather_kernel(x2d, src)


def _scale_matmul(sums, inv, w_big, b_big, block_n=512):
    n, fv = sums.shape

    def body(s_ref, c_ref, w_ref, b_ref, o_ref):
        o_ref[...] = (
            jnp.dot(s_ref[...] * c_ref[...], w_ref[...],
                    preferred_element_type=jnp.float32)
            + b_ref[...]
        )

    return pl.pallas_call(
        body,
        grid=(n // block_n,),
        in_specs=[
            pl.BlockSpec((block_n, fv), lambda i: (i, 0)),
            pl.BlockSpec((block_n, 1), lambda i: (i, 0)),
            pl.BlockSpec((fv, fv), lambda i: (0, 0)),
            pl.BlockSpec((1, fv), lambda i: (0, 0)),
        ],
        out_specs=pl.BlockSpec((block_n, fv), lambda i: (i, 0)),
        out_shape=jax.ShapeDtypeStruct((n, fv), jnp.float32),
    )(sums, inv, w_big, b_big)


def kernel(x, edge, pool_edge, finer_edge, W, b):
    node_num, f_dim, v_dim = x.shape
    fv = f_dim * v_dim
    finer_nodes = 2 * node_num

    fv_pad = ((fv + 127) // 128) * 128
    x2d = jnp.pad(x.reshape(node_num, fv), ((0, 0), (0, fv_pad - fv)))
    src = edge[pool_edge].reshape(-1)
    B = src.shape[0]
    nw_ch = 32 * _CH
    B_pad = ((B + nw_ch - 1) // nw_ch) * nw_ch
    src_p = jnp.concatenate(
        [src, jnp.zeros((B_pad - B,), dtype=jnp.int32)])

    gathered = _sc_gather(x2d, src_p)[:B]

    dst = finer_edge.reshape(-1)
    sums = jax.ops.segment_sum(
        gathered, dst, num_segments=finer_nodes)[:, :fv]
    counts = jax.ops.segment_sum(
        jnp.ones((B,), dtype=jnp.float32), dst, num_segments=finer_nodes)
    inv = (1.0 / jnp.maximum(counts, 1.0))[:, None]

    block_n = 512
    n_pad = ((finer_nodes + block_n - 1) // block_n) * block_n
    sums_p = jnp.pad(sums, ((0, n_pad - finer_nodes), (0, 0)))
    inv_p = jnp.pad(inv, ((0, n_pad - finer_nodes), (0, 0)))

    w_big = jnp.kron(W.T, jnp.eye(v_dim, dtype=jnp.float32))
    b_big = jnp.repeat(b, v_dim)[None, :]

    out = _scale_matmul(sums_p, inv_p, w_big, b_big, block_n=block_n)
    return out[:finer_nodes].reshape(finer_nodes, f_dim, v_dim)


# speedup vs baseline: 2.9585x; 1.1023x over previous
import functools

import jax
import jax.numpy as jnp
from jax import lax
from jax.experimental import pallas as pl
from jax.experimental.pallas import tpu as pltpu
from jax.experimental.pallas import tpu_sc as plsc

_CH = 128


def _sc_gather(x2d, src):
    B = src.shape[0]
    fv = x2d.shape[1]
    info = plsc.get_sparse_core_info()
    nw = info.num_cores * info.num_subcores
    per_w = B // nw
    n_ch = per_w // _CH
    mesh = plsc.VectorSubcoreMesh(core_axis_name="c", subcore_axis_name="s")

    @functools.partial(
        pl.kernel,
        mesh=mesh,
        out_type=jax.ShapeDtypeStruct((B, fv), jnp.float32),
        scratch_types=[
            pltpu.VMEM((_CH,), jnp.int32),
            pltpu.VMEM((_CH, fv), jnp.float32),
            pltpu.SemaphoreType.DMA,
        ],
    )
    def gather_kernel(x_hbm, src_hbm, out_hbm, idx_v, rows_v, sem):
        wid = lax.axis_index("s") * info.num_cores + lax.axis_index("c")
        base = wid * per_w

        def body(i, carry):
            off = base + i * _CH
            pltpu.sync_copy(src_hbm.at[pl.ds(off, _CH)], idx_v)
            pltpu.async_copy(x_hbm.at[idx_v], rows_v, sem).wait()
            pltpu.sync_copy(rows_v, out_hbm.at[pl.ds(off, _CH)])
            return carry

        lax.fori_loop(0, n_ch, body, 0)

    return gather_kernel(x2d, src)


def _scale_matmul(sums, inv, w_big, b_big, block_n=400):
    n, fvp = sums.shape
    fv = w_big.shape[1]

    def body(s_ref, c_ref, w_ref, b_ref, o_ref):
        o_ref[...] = (
            jnp.dot(s_ref[...] * c_ref[...], w_ref[...],
                    preferred_element_type=jnp.float32)
            + b_ref[...]
        )

    return pl.pallas_call(
        body,
        grid=(n // block_n,),
        in_specs=[
            pl.BlockSpec((block_n, fvp), lambda i: (i, 0)),
            pl.BlockSpec((block_n, 1), lambda i: (i, 0)),
            pl.BlockSpec((fvp, fv), lambda i: (0, 0)),
            pl.BlockSpec((1, fv), lambda i: (0, 0)),
        ],
        out_specs=pl.BlockSpec((block_n, fv), lambda i: (i, 0)),
        out_shape=jax.ShapeDtypeStruct((n, fv), jnp.float32),
    )(sums, inv, w_big, b_big)


def kernel(x, edge, pool_edge, finer_edge, W, b):
    node_num, f_dim, v_dim = x.shape
    fv = f_dim * v_dim
    finer_nodes = 2 * node_num

    fv_pad = ((fv + 127) // 128) * 128
    x2d = jnp.pad(x.reshape(node_num, fv), ((0, 0), (0, fv_pad - fv)))
    src = edge[pool_edge].reshape(-1)
    B = src.shape[0]
    nw_ch = 32 * _CH
    B_pad = ((B + nw_ch - 1) // nw_ch) * nw_ch
    src_p = jnp.concatenate(
        [src, jnp.zeros((B_pad - B,), dtype=jnp.int32)])

    gathered = _sc_gather(x2d, src_p)[:B]

    dst = finer_edge.reshape(-1)
    sums = jax.ops.segment_sum(gathered, dst, num_segments=finer_nodes)
    counts = jax.ops.segment_sum(
        jnp.ones((B,), dtype=jnp.float32), dst, num_segments=finer_nodes)
    inv = (1.0 / jnp.maximum(counts, 1.0))[:, None]

    w_big = jnp.pad(jnp.kron(W.T, jnp.eye(v_dim, dtype=jnp.float32)),
                    ((0, fv_pad - fv), (0, 0)))
    b_big = jnp.repeat(b, v_dim)[None, :]

    out = _scale_matmul(sums, inv, w_big, b_big, block_n=400)
    return out.reshape(finer_nodes, f_dim, v_dim)
